# baseline (speedup 1.0000x reference)
import jax
import jax.numpy as jnp
from jax.experimental import pallas as pl
from jax.experimental.pallas import tpu as pltpu

N_CLASSES = 64
K_NEIGHBORS = 16
ITERATIONS = 3
N = 8192
NS = 4096
BLK = 256
NB = N // BLK


def _knn_body(emb_blk_ref, emb_ref, sq_blk_ref, sq_row_ref, idx_ref):
    dot = jax.lax.dot_general(
        emb_blk_ref[...], emb_ref[...], (((1,), (1,)), ((), ())),
        precision=None)
    d2 = sq_blk_ref[...] + sq_row_ref[...] - 2.0 * dot
    col = jax.lax.broadcasted_iota(jnp.int32, (BLK, N), 1)
    picks = []
    for _ in range(K_NEIGHBORS):
        m = jnp.min(d2, axis=1, keepdims=True)
        j = jnp.min(jnp.where(d2 == m, col, N), axis=1, keepdims=True)
        picks.append(j)
        d2 = jnp.where(col == j, jnp.inf, d2)
    idx_ref[...] = jnp.concatenate(picks, axis=1)


def _adj_body(idx_blk_ref, idxt_ref, adj_ref, deg_ref):
    col = jax.lax.broadcasted_iota(jnp.int32, (BLK, N), 1)
    row = jax.lax.broadcasted_iota(jnp.int32, (BLK, N), 0)
    base = pl.program_id(0) * BLK
    acc = jnp.zeros((BLK, N), jnp.bool_)
    for k in range(K_NEIGHBORS):
        acc = acc | (col == idx_blk_ref[:, k][:, None])
        acc = acc | (idxt_ref[k, :][None, :] == row + base)
    a = acc.astype(jnp.float32)
    adj_ref[...] = a
    deg_ref[...] = jnp.sum(a, axis=1, keepdims=True)


def _prop_body(labels_ref, adj_ref, deg_ref, out_ref, x_ref):
    t = pl.program_id(0)
    b = pl.program_id(1)

    @pl.when(jnp.logical_and(t == 0, b == 0))
    def _init():
        lbl = labels_ref[...]
        cls = jax.lax.broadcasted_iota(jnp.int32, (N, N_CLASSES), 1)
        rid = jax.lax.broadcasted_iota(jnp.int32, (N, N_CLASSES), 0)
        x_ref[0] = ((cls == lbl) & (rid < NS)).astype(jnp.float32)

    cur = jax.lax.rem(t, 2)
    nxt = jax.lax.rem(t + 1, 2)
    trans = adj_ref[...] * (1.0 / deg_ref[...])
    y = jax.lax.dot(trans, x_ref[cur], precision=None)

    @pl.when(t < ITERATIONS - 1)
    def _store():
        x_ref[nxt, pl.ds(b * BLK, BLK), :] = y

    @pl.when(jnp.logical_and(t == ITERATIONS - 1, b >= NB // 2))
    def _finish():
        cls = jax.lax.broadcasted_iota(jnp.int32, (BLK, N_CLASSES), 1)
        m = jnp.max(y, axis=1, keepdims=True)
        am = jnp.min(jnp.where(y == m, cls, N_CLASSES), axis=1, keepdims=True)
        out_ref[0] = (cls == am).astype(jnp.float32)


def kernel(support, query, support_labels):
    b, n_support, d = support.shape
    emb = jnp.concatenate(
        [support.reshape(-1, d), query.reshape(-1, d)], axis=0)
    sq = jnp.sum(emb * emb, axis=1)
    labels_pad = jnp.pad(support_labels, (0, N - n_support),
                         constant_values=-1)[:, None]

    idx = pl.pallas_call(
        _knn_body,
        grid=(NB,),
        in_specs=[
            pl.BlockSpec((BLK, d), lambda i: (i, 0)),
            pl.BlockSpec((N, d), lambda i: (0, 0)),
            pl.BlockSpec((BLK, 1), lambda i: (i, 0)),
            pl.BlockSpec((1, N), lambda i: (0, 0)),
        ],
        out_specs=pl.BlockSpec((BLK, K_NEIGHBORS), lambda i: (i, 0)),
        out_shape=jax.ShapeDtypeStruct((N, K_NEIGHBORS), jnp.int32),
    )(emb, emb, sq[:, None], sq[None, :])

    adj, deg = pl.pallas_call(
        _adj_body,
        grid=(NB,),
        in_specs=[
            pl.BlockSpec((BLK, K_NEIGHBORS), lambda i: (i, 0)),
            pl.BlockSpec((K_NEIGHBORS, N), lambda i: (0, 0)),
        ],
        out_specs=[
            pl.BlockSpec((BLK, N), lambda i: (i, 0)),
            pl.BlockSpec((BLK, 1), lambda i: (i, 0)),
        ],
        out_shape=[
            jax.ShapeDtypeStruct((N, N), jnp.float32),
            jax.ShapeDtypeStruct((N, 1), jnp.float32),
        ],
    )(idx, idx.T)

    logits = pl.pallas_call(
        _prop_body,
        grid=(ITERATIONS, NB),
        in_specs=[
            pl.BlockSpec((N, 1), lambda t, i: (0, 0)),
            pl.BlockSpec((BLK, N), lambda t, i: (i, 0)),
            pl.BlockSpec((BLK, 1), lambda t, i: (i, 0)),
        ],
        out_specs=pl.BlockSpec(
            (1, BLK, N_CLASSES),
            lambda t, i: (t, jnp.maximum(i - NB // 2, 0), 0)),
        out_shape=jax.ShapeDtypeStruct(
            (ITERATIONS, N - NS, N_CLASSES), jnp.float32),
        scratch_shapes=[pltpu.VMEM((2, N, N_CLASSES), jnp.float32)],
    )(labels_pad, adj, deg)

    return logits[ITERATIONS - 1].reshape(1, N - NS, N_CLASSES)
